# Initial kernel scaffold; baseline (speedup 1.0000x reference)
import jax
import jax.numpy as jnp
from jax.experimental import pallas as pl


def kernel(inputs, adj, W_emb, b_emb, W_ru, b_ru, W_c, b_c):
    raise NotImplementedError("write your pallas kernel here")


# speedup vs baseline: 1.2444x; 1.2444x over previous
import jax
import jax.numpy as jnp
from jax.experimental import pallas as pl
from jax.experimental.pallas import tpu as pltpu

B = 4
N = 2048
WINDOW = 12
EMB = 32
HID = 32
K = 2
DIN = EMB + HID
FB = B * EMB
TILE = 256
NT = N // TILE


def _dcgru_body(xin_ref, adj_ref, wemb_ref, wuc_ref, buc_ref, out_ref,
                x_sc, x1_sc):
    k = pl.program_id(0)
    i = pl.program_id(1)

    @pl.when((k == 0) & (i == 0))
    def _embed():
        x_sc[...] = (jnp.dot(xin_ref[...], wemb_ref[...],
                             preferred_element_type=jnp.float32)
                     + buc_ref[0:1, 2 * FB:3 * FB])

    @pl.when(k == 0)
    def _pass1():
        a = adj_ref[...]
        rs = jnp.sum(a, axis=1, keepdims=True) + 1e-8
        y = jnp.dot(a, x_sc[...], preferred_element_type=jnp.float32)
        x1_sc[pl.ds(i * TILE, TILE), :] = y / rs

    @pl.when(k == 1)
    def _pass2():
        a = adj_ref[...]
        rs = jnp.sum(a, axis=1, keepdims=True) + 1e-8
        x2 = jnp.dot(a, x1_sc[...], preferred_element_type=jnp.float32) / rs
        xt = x_sc[pl.ds(i * TILE, TILE), :]
        x1t = x1_sc[pl.ds(i * TILE, TILE), :]
        uc = (jnp.dot(xt, wuc_ref[0:FB, :], preferred_element_type=jnp.float32)
              + jnp.dot(x1t, wuc_ref[FB:2 * FB, :],
                        preferred_element_type=jnp.float32)
              + jnp.dot(x2, wuc_ref[2 * FB:3 * FB, :],
                        preferred_element_type=jnp.float32))
        u = jax.nn.sigmoid(uc[:, 0:FB] + buc_ref[0:1, 0:FB])
        c = jnp.tanh(uc[:, FB:2 * FB] + buc_ref[0:1, FB:2 * FB])
        out_ref[...] = (1.0 - u) * c


def kernel(inputs, adj, W_emb, b_emb, W_ru, b_ru, W_c, b_c):
    xin = inputs.transpose(1, 0, 2).reshape(N, B * WINDOW)
    eye_b = jnp.eye(B, dtype=jnp.float32)
    wemb = jnp.kron(eye_b, W_emb)
    wuc = jnp.concatenate([
        jnp.concatenate([jnp.kron(eye_b, W_ru[DIN * s:DIN * s + EMB, HID:2 * HID]),
                         jnp.kron(eye_b, W_c[DIN * s:DIN * s + EMB, 0:HID])],
                        axis=1)
        for s in range(K + 1)], axis=0)
    buc = jnp.tile(jnp.concatenate([jnp.tile(b_ru[HID:2 * HID], B),
                                    jnp.tile(b_c, B),
                                    jnp.tile(b_emb, B)]).reshape(1, 3 * FB),
                   (8, 1))

    out = pl.pallas_call(
        _dcgru_body,
        grid=(2, NT),
        in_specs=[
            pl.BlockSpec((N, B * WINDOW), lambda k, i: (0, 0)),
            pl.BlockSpec((TILE, N), lambda k, i: (i, 0)),
            pl.BlockSpec((B * WINDOW, FB), lambda k, i: (0, 0)),
            pl.BlockSpec((3 * FB, 2 * FB), lambda k, i: (0, 0)),
            pl.BlockSpec((8, 3 * FB), lambda k, i: (0, 0)),
        ],
        out_specs=pl.BlockSpec((TILE, FB), lambda k, i: (i, 0)),
        out_shape=jax.ShapeDtypeStruct((N, FB), jnp.float32),
        scratch_shapes=[
            pltpu.VMEM((N, FB), jnp.float32),
            pltpu.VMEM((N, FB), jnp.float32),
        ],
        compiler_params=pltpu.CompilerParams(
            dimension_semantics=("arbitrary", "arbitrary")),
    )(xin, adj, wemb, wuc, buc)
    return out.reshape(N, B, HID).transpose(1, 0, 2)


# speedup vs baseline: 1.3322x; 1.0706x over previous
import jax
import jax.numpy as jnp
from jax.experimental import pallas as pl
from jax.experimental.pallas import tpu as pltpu

B = 4
N = 2048
WINDOW = 12
EMB = 32
HID = 32
K = 2
DIN = EMB + HID
FB = B * EMB
TILE = 256
NT = N // TILE


def _dcgru_body(xin_ref, adj_ref, wemb_ref, wuc_ref, buc_ref, out_ref,
                x_sc, x1_sc, adj_sc):
    k = pl.program_id(0)
    i = pl.program_id(1)

    @pl.when((k == 0) & (i == 0))
    def _embed():
        x_sc[...] = (jnp.dot(xin_ref[...], wemb_ref[...],
                             preferred_element_type=jnp.float32)
                     + buc_ref[0:1, 2 * FB:3 * FB])

    @pl.when(k == 0)
    def _pass1():
        a = adj_ref[...]
        adj_sc[pl.ds(i * TILE, TILE), :] = a
        rs = jnp.sum(a, axis=1, keepdims=True) + 1e-8
        y = jnp.dot(a, x_sc[...], preferred_element_type=jnp.float32)
        x1_sc[pl.ds(i * TILE, TILE), :] = y / rs

    @pl.when(k == 1)
    def _pass2():
        a = adj_sc[pl.ds(i * TILE, TILE), :]
        rs = jnp.sum(a, axis=1, keepdims=True) + 1e-8
        x2 = jnp.dot(a, x1_sc[...], preferred_element_type=jnp.float32) / rs
        xt = x_sc[pl.ds(i * TILE, TILE), :]
        x1t = x1_sc[pl.ds(i * TILE, TILE), :]
        uc = (jnp.dot(xt, wuc_ref[0:FB, :], preferred_element_type=jnp.float32)
              + jnp.dot(x1t, wuc_ref[FB:2 * FB, :],
                        preferred_element_type=jnp.float32)
              + jnp.dot(x2, wuc_ref[2 * FB:3 * FB, :],
                        preferred_element_type=jnp.float32))
        u = jax.nn.sigmoid(uc[:, 0:FB] + buc_ref[0:1, 0:FB])
        c = jnp.tanh(uc[:, FB:2 * FB] + buc_ref[0:1, FB:2 * FB])
        out_ref[...] = (1.0 - u) * c


def kernel(inputs, adj, W_emb, b_emb, W_ru, b_ru, W_c, b_c):
    xin = inputs.transpose(1, 0, 2).reshape(N, B * WINDOW)
    eye_b = jnp.eye(B, dtype=jnp.float32)
    wemb = jnp.kron(eye_b, W_emb)
    wuc = jnp.concatenate([
        jnp.concatenate([jnp.kron(eye_b, W_ru[DIN * s:DIN * s + EMB, HID:2 * HID]),
                         jnp.kron(eye_b, W_c[DIN * s:DIN * s + EMB, 0:HID])],
                        axis=1)
        for s in range(K + 1)], axis=0)
    buc = jnp.tile(jnp.concatenate([jnp.tile(b_ru[HID:2 * HID], B),
                                    jnp.tile(b_c, B),
                                    jnp.tile(b_emb, B)]).reshape(1, 3 * FB),
                   (8, 1))

    out = pl.pallas_call(
        _dcgru_body,
        grid=(2, NT),
        in_specs=[
            pl.BlockSpec((N, B * WINDOW), lambda k, i: (0, 0)),
            pl.BlockSpec((TILE, N), lambda k, i: (i * (1 - k), 0)),
            pl.BlockSpec((B * WINDOW, FB), lambda k, i: (0, 0)),
            pl.BlockSpec((3 * FB, 2 * FB), lambda k, i: (0, 0)),
            pl.BlockSpec((8, 3 * FB), lambda k, i: (0, 0)),
        ],
        out_specs=pl.BlockSpec((TILE, FB), lambda k, i: (i, 0)),
        out_shape=jax.ShapeDtypeStruct((N, FB), jnp.float32),
        scratch_shapes=[
            pltpu.VMEM((N, FB), jnp.float32),
            pltpu.VMEM((N, FB), jnp.float32),
            pltpu.VMEM((N, N), jnp.float32),
        ],
        compiler_params=pltpu.CompilerParams(
            dimension_semantics=("arbitrary", "arbitrary")),
    )(xin, adj, wemb, wuc, buc)
    return out.reshape(N, B, HID).transpose(1, 0, 2)


# speedup vs baseline: 1.3452x; 1.0098x over previous
import jax
import jax.numpy as jnp
from jax.experimental import pallas as pl
from jax.experimental.pallas import tpu as pltpu

B = 4
N = 2048
WINDOW = 12
EMB = 32
HID = 32
K = 2
DIN = EMB + HID
FB = B * EMB
TILE = 256
NT = N // TILE


def _dcgru_body(xin_ref, adj_ref, wemb_ref, wuc_ref, buc_ref, out_ref,
                x_sc, x1_sc, adj_sc, xb_sc, x1b_sc):
    k = pl.program_id(0)
    i = pl.program_id(1)

    @pl.when((k == 0) & (i == 0))
    def _embed():
        x = (jnp.dot(xin_ref[...], wemb_ref[...],
                     preferred_element_type=jnp.float32)
             + buc_ref[0:1, 2 * FB:3 * FB])
        x_sc[...] = x
        xb_sc[...] = x.astype(jnp.bfloat16)

    @pl.when(k == 0)
    def _pass1():
        a = adj_ref[...]
        ab = a.astype(jnp.bfloat16)
        adj_sc[pl.ds(i * TILE, TILE), :] = ab
        rs = jnp.sum(a, axis=1, keepdims=True) + 1e-8
        y = jnp.dot(ab, xb_sc[...], preferred_element_type=jnp.float32)
        x1 = y / rs
        x1_sc[pl.ds(i * TILE, TILE), :] = x1
        x1b_sc[pl.ds(i * TILE, TILE), :] = x1.astype(jnp.bfloat16)

    @pl.when(k == 1)
    def _pass2():
        a = adj_sc[pl.ds(i * TILE, TILE), :]
        rs = (jnp.sum(a.astype(jnp.float32), axis=1, keepdims=True) + 1e-8)
        x2 = jnp.dot(a, x1b_sc[...], preferred_element_type=jnp.float32) / rs
        xt = x_sc[pl.ds(i * TILE, TILE), :]
        x1t = x1_sc[pl.ds(i * TILE, TILE), :]
        uc = (jnp.dot(xt, wuc_ref[0:FB, :], preferred_element_type=jnp.float32)
              + jnp.dot(x1t, wuc_ref[FB:2 * FB, :],
                        preferred_element_type=jnp.float32)
              + jnp.dot(x2, wuc_ref[2 * FB:3 * FB, :],
                        preferred_element_type=jnp.float32))
        u = jax.nn.sigmoid(uc[:, 0:FB] + buc_ref[0:1, 0:FB])
        c = jnp.tanh(uc[:, FB:2 * FB] + buc_ref[0:1, FB:2 * FB])
        out_ref[...] = (1.0 - u) * c


def kernel(inputs, adj, W_emb, b_emb, W_ru, b_ru, W_c, b_c):
    xin = inputs.transpose(1, 0, 2).reshape(N, B * WINDOW)
    eye_b = jnp.eye(B, dtype=jnp.float32)
    wemb = jnp.kron(eye_b, W_emb)
    wuc = jnp.concatenate([
        jnp.concatenate([jnp.kron(eye_b, W_ru[DIN * s:DIN * s + EMB, HID:2 * HID]),
                         jnp.kron(eye_b, W_c[DIN * s:DIN * s + EMB, 0:HID])],
                        axis=1)
        for s in range(K + 1)], axis=0)
    buc = jnp.tile(jnp.concatenate([jnp.tile(b_ru[HID:2 * HID], B),
                                    jnp.tile(b_c, B),
                                    jnp.tile(b_emb, B)]).reshape(1, 3 * FB),
                   (8, 1))

    out = pl.pallas_call(
        _dcgru_body,
        grid=(2, NT),
        in_specs=[
            pl.BlockSpec((N, B * WINDOW), lambda k, i: (0, 0)),
            pl.BlockSpec((TILE, N), lambda k, i: (i * (1 - k), 0)),
            pl.BlockSpec((B * WINDOW, FB), lambda k, i: (0, 0)),
            pl.BlockSpec((3 * FB, 2 * FB), lambda k, i: (0, 0)),
            pl.BlockSpec((8, 3 * FB), lambda k, i: (0, 0)),
        ],
        out_specs=pl.BlockSpec((TILE, FB), lambda k, i: (i, 0)),
        out_shape=jax.ShapeDtypeStruct((N, FB), jnp.float32),
        scratch_shapes=[
            pltpu.VMEM((N, FB), jnp.float32),
            pltpu.VMEM((N, FB), jnp.float32),
            pltpu.VMEM((N, N), jnp.bfloat16),
            pltpu.VMEM((N, FB), jnp.bfloat16),
            pltpu.VMEM((N, FB), jnp.bfloat16),
        ],
        compiler_params=pltpu.CompilerParams(
            dimension_semantics=("arbitrary", "arbitrary")),
    )(xin, adj, wemb, wuc, buc)
    return out.reshape(N, B, HID).transpose(1, 0, 2)


# speedup vs baseline: 1.3549x; 1.0073x over previous
import jax
import jax.numpy as jnp
from jax.experimental import pallas as pl
from jax.experimental.pallas import tpu as pltpu

B = 4
N = 2048
WINDOW = 12
EMB = 32
HID = 32
K = 2
DIN = EMB + HID
FB = B * EMB
TILE = 256
NT = N // TILE


def _dcgru_body(inp_ref, adj_ref, wemb_ref, wru_ref, wc_ref,
                bemb_ref, bru_ref, bc_ref, out_ref,
                x_sc, x1_sc, adj_sc, xb_sc, x1b_sc, wuc_sc):
    k = pl.program_id(0)
    i = pl.program_id(1)

    @pl.when((k == 0) & (i == 0))
    def _setup():
        wuc_sc[...] = jnp.zeros_like(wuc_sc)
        for s in range(K + 1):
            for b in range(B):
                r0 = FB * s + EMB * b
                wuc_sc[r0:r0 + EMB, HID * b:HID * b + HID] = (
                    wru_ref[DIN * s:DIN * s + EMB, HID:2 * HID])
                wuc_sc[r0:r0 + EMB, FB + HID * b:FB + HID * b + HID] = (
                    wc_ref[DIN * s:DIN * s + EMB, 0:HID])
        for b in range(B):
            x_sc[:, EMB * b:EMB * b + EMB] = (
                jnp.dot(inp_ref[b], wemb_ref[...],
                        preferred_element_type=jnp.float32)
                + bemb_ref[...])
        xb_sc[...] = x_sc[...].astype(jnp.bfloat16)

    @pl.when(k == 0)
    def _pass1():
        a = adj_ref[...]
        ab = a.astype(jnp.bfloat16)
        adj_sc[pl.ds(i * TILE, TILE), :] = ab
        rs = jnp.sum(a, axis=1, keepdims=True) + 1e-8
        y = jnp.dot(ab, xb_sc[...], preferred_element_type=jnp.float32)
        x1 = y / rs
        x1_sc[pl.ds(i * TILE, TILE), :] = x1
        x1b_sc[pl.ds(i * TILE, TILE), :] = x1.astype(jnp.bfloat16)

    @pl.when(k == 1)
    def _pass2():
        a = adj_sc[pl.ds(i * TILE, TILE), :]
        rs = (jnp.sum(a.astype(jnp.float32), axis=1, keepdims=True) + 1e-8)
        x2 = jnp.dot(a, x1b_sc[...], preferred_element_type=jnp.float32) / rs
        xt = x_sc[pl.ds(i * TILE, TILE), :]
        x1t = x1_sc[pl.ds(i * TILE, TILE), :]
        uc = (jnp.dot(xt, wuc_sc[0:FB, :], preferred_element_type=jnp.float32)
              + jnp.dot(x1t, wuc_sc[FB:2 * FB, :],
                        preferred_element_type=jnp.float32)
              + jnp.dot(x2, wuc_sc[2 * FB:3 * FB, :],
                        preferred_element_type=jnp.float32))
        bu = jnp.concatenate([bru_ref[0:1, HID:2 * HID]] * B, axis=1)
        bc = jnp.concatenate([bc_ref[0:1, :]] * B, axis=1)
        u = jax.nn.sigmoid(uc[:, 0:FB] + bu)
        c = jnp.tanh(uc[:, FB:2 * FB] + bc)
        o = (1.0 - u) * c
        for b in range(B):
            out_ref[b, :, :] = o[:, HID * b:HID * b + HID]


def kernel(inputs, adj, W_emb, b_emb, W_ru, b_ru, W_c, b_c):
    return pl.pallas_call(
        _dcgru_body,
        grid=(2, NT),
        in_specs=[
            pl.BlockSpec((B, N, WINDOW), lambda k, i: (0, 0, 0)),
            pl.BlockSpec((TILE, N), lambda k, i: (i * (1 - k), 0)),
            pl.BlockSpec((WINDOW, EMB), lambda k, i: (0, 0)),
            pl.BlockSpec(((K + 1) * DIN, 2 * HID), lambda k, i: (0, 0)),
            pl.BlockSpec(((K + 1) * DIN, HID), lambda k, i: (0, 0)),
            pl.BlockSpec((1, EMB), lambda k, i: (0, 0)),
            pl.BlockSpec((1, 2 * HID), lambda k, i: (0, 0)),
            pl.BlockSpec((1, HID), lambda k, i: (0, 0)),
        ],
        out_specs=pl.BlockSpec((B, TILE, HID), lambda k, i: (0, i, 0)),
        out_shape=jax.ShapeDtypeStruct((B, N, HID), jnp.float32),
        scratch_shapes=[
            pltpu.VMEM((N, FB), jnp.float32),
            pltpu.VMEM((N, FB), jnp.float32),
            pltpu.VMEM((N, N), jnp.bfloat16),
            pltpu.VMEM((N, FB), jnp.bfloat16),
            pltpu.VMEM((N, FB), jnp.bfloat16),
            pltpu.VMEM((3 * FB, 2 * FB), jnp.float32),
        ],
        compiler_params=pltpu.CompilerParams(
            dimension_semantics=("arbitrary", "arbitrary")),
    )(inputs, adj, W_emb, W_ru, W_c,
      b_emb.reshape(1, EMB), b_ru.reshape(1, 2 * HID), b_c.reshape(1, HID))


# speedup vs baseline: 1.3561x; 1.0009x over previous
import jax
import jax.numpy as jnp
from jax.experimental import pallas as pl
from jax.experimental.pallas import tpu as pltpu

B = 4
N = 2048
WINDOW = 12
EMB = 32
HID = 32
K = 2
DIN = EMB + HID
FB = B * EMB
TILE = 256
NT = N // TILE


def _dcgru_body(inp_ref, adj_ref, wemb_ref, wru_ref, wc_ref,
                bemb_ref, bru_ref, bc_ref, out_ref,
                x_sc, x1_sc, adj_sc, xb_sc, x1b_sc, wuc_sc):
    k = pl.program_id(0)
    i = pl.program_id(1)

    @pl.when((k == 0) & (i == 0))
    def _setup():
        wuc_sc[...] = jnp.zeros_like(wuc_sc)
        for s in range(K + 1):
            for b in range(B):
                r0 = FB * s + EMB * b
                wuc_sc[r0:r0 + EMB, HID * b:HID * b + HID] = (
                    wru_ref[DIN * s:DIN * s + EMB, HID:2 * HID])
                wuc_sc[r0:r0 + EMB, FB + HID * b:FB + HID * b + HID] = (
                    wc_ref[DIN * s:DIN * s + EMB, 0:HID])
        for b in range(B):
            x_sc[:, EMB * b:EMB * b + EMB] = (
                jnp.dot(inp_ref[b], wemb_ref[...],
                        preferred_element_type=jnp.float32)
                + bemb_ref[...].reshape(1, EMB))
        xb_sc[...] = x_sc[...].astype(jnp.bfloat16)

    @pl.when(k == 0)
    def _pass1():
        a = adj_ref[...]
        ab = a.astype(jnp.bfloat16)
        adj_sc[pl.ds(i * TILE, TILE), :] = ab
        rs = jnp.sum(a, axis=1, keepdims=True) + 1e-8
        y = jnp.dot(ab, xb_sc[...], preferred_element_type=jnp.float32)
        x1 = y / rs
        x1_sc[pl.ds(i * TILE, TILE), :] = x1
        x1b_sc[pl.ds(i * TILE, TILE), :] = x1.astype(jnp.bfloat16)

    @pl.when(k == 1)
    def _pass2():
        a = adj_sc[pl.ds(i * TILE, TILE), :]
        rs = (jnp.sum(a.astype(jnp.float32), axis=1, keepdims=True) + 1e-8)
        x2 = jnp.dot(a, x1b_sc[...], preferred_element_type=jnp.float32) / rs
        xt = x_sc[pl.ds(i * TILE, TILE), :]
        x1t = x1_sc[pl.ds(i * TILE, TILE), :]
        uc = (jnp.dot(xt, wuc_sc[0:FB, :], preferred_element_type=jnp.float32)
              + jnp.dot(x1t, wuc_sc[FB:2 * FB, :],
                        preferred_element_type=jnp.float32)
              + jnp.dot(x2, wuc_sc[2 * FB:3 * FB, :],
                        preferred_element_type=jnp.float32))
        bu = jnp.concatenate([bru_ref[HID:2 * HID].reshape(1, HID)] * B,
                             axis=1)
        bc = jnp.concatenate([bc_ref[...].reshape(1, HID)] * B, axis=1)
        u = jax.nn.sigmoid(uc[:, 0:FB] + bu)
        c = jnp.tanh(uc[:, FB:2 * FB] + bc)
        o = (1.0 - u) * c
        for b in range(B):
            out_ref[b, :, :] = o[:, HID * b:HID * b + HID]


def kernel(inputs, adj, W_emb, b_emb, W_ru, b_ru, W_c, b_c):
    return pl.pallas_call(
        _dcgru_body,
        grid=(2, NT),
        in_specs=[
            pl.BlockSpec((B, N, WINDOW), lambda k, i: (0, 0, 0)),
            pl.BlockSpec((TILE, N), lambda k, i: (i * (1 - k), 0)),
            pl.BlockSpec((WINDOW, EMB), lambda k, i: (0, 0)),
            pl.BlockSpec(((K + 1) * DIN, 2 * HID), lambda k, i: (0, 0)),
            pl.BlockSpec(((K + 1) * DIN, HID), lambda k, i: (0, 0)),
            pl.BlockSpec((EMB,), lambda k, i: (0,)),
            pl.BlockSpec((2 * HID,), lambda k, i: (0,)),
            pl.BlockSpec((HID,), lambda k, i: (0,)),
        ],
        out_specs=pl.BlockSpec((B, TILE, HID), lambda k, i: (0, i, 0)),
        out_shape=jax.ShapeDtypeStruct((B, N, HID), jnp.float32),
        scratch_shapes=[
            pltpu.VMEM((N, FB), jnp.float32),
            pltpu.VMEM((N, FB), jnp.float32),
            pltpu.VMEM((N, N), jnp.bfloat16),
            pltpu.VMEM((N, FB), jnp.bfloat16),
            pltpu.VMEM((N, FB), jnp.bfloat16),
            pltpu.VMEM((3 * FB, 2 * FB), jnp.float32),
        ],
        compiler_params=pltpu.CompilerParams(
            dimension_semantics=("arbitrary", "arbitrary")),
    )(inputs, adj, W_emb, W_ru, W_c, b_emb, b_ru, b_c)


# speedup vs baseline: 1.8430x; 1.3590x over previous
import jax
import jax.numpy as jnp
from jax.experimental import pallas as pl
from jax.experimental.pallas import tpu as pltpu

B = 4
N = 2048
WINDOW = 12
EMB = 32
HID = 32
K = 2
DIN = EMB + HID
FB = B * EMB
TILE = 256
NT = N // TILE


def _dcgru_body(inp_ref, adj_ref, wemb_ref, wrut_ref, wct_ref,
                bemb_ref, bru_ref, bc_ref, out_ref,
                x_sc, x1_sc, adj_sc, xb_sc, x1b_sc, wuc_sc):
    k = pl.program_id(0)
    i = pl.program_id(1)

    @pl.when((k == 0) & (i == 0))
    def _setup():
        wuc_sc[...] = jnp.zeros_like(wuc_sc)
        for s in range(K + 1):
            for b in range(B):
                r0 = FB * s + EMB * b
                wuc_sc[r0:r0 + EMB, HID * b:HID * b + HID] = (
                    wrut_ref[HID:2 * HID, DIN * s:DIN * s + EMB].T)
                wuc_sc[r0:r0 + EMB, FB + HID * b:FB + HID * b + HID] = (
                    wct_ref[:, DIN * s:DIN * s + EMB].T)
        for b in range(B):
            xtb = jax.lax.dot_general(
                wemb_ref[...], inp_ref[:, b, :],
                (((0,), (0,)), ((), ())),
                preferred_element_type=jnp.float32)
            x_sc[:, EMB * b:EMB * b + EMB] = (
                xtb.T + bemb_ref[...].reshape(1, EMB))
        xb_sc[...] = x_sc[...].astype(jnp.bfloat16)

    @pl.when(k == 0)
    def _pass1():
        a = adj_ref[...]
        ab = a.astype(jnp.bfloat16)
        adj_sc[pl.ds(i * TILE, TILE), :] = ab
        rs = jnp.sum(a, axis=1, keepdims=True) + 1e-8
        y = jnp.dot(ab, xb_sc[...], preferred_element_type=jnp.float32)
        x1 = y / rs
        x1_sc[pl.ds(i * TILE, TILE), :] = x1
        x1b_sc[pl.ds(i * TILE, TILE), :] = x1.astype(jnp.bfloat16)

    @pl.when(k == 1)
    def _pass2():
        a = adj_sc[pl.ds(i * TILE, TILE), :]
        rs = (jnp.sum(a.astype(jnp.float32), axis=1, keepdims=True) + 1e-8)
        x2 = jnp.dot(a, x1b_sc[...], preferred_element_type=jnp.float32) / rs
        xt = x_sc[pl.ds(i * TILE, TILE), :]
        x1t = x1_sc[pl.ds(i * TILE, TILE), :]
        uc = (jnp.dot(xt, wuc_sc[0:FB, :], preferred_element_type=jnp.float32)
              + jnp.dot(x1t, wuc_sc[FB:2 * FB, :],
                        preferred_element_type=jnp.float32)
              + jnp.dot(x2, wuc_sc[2 * FB:3 * FB, :],
                        preferred_element_type=jnp.float32))
        bu = jnp.concatenate([bru_ref[HID:2 * HID].reshape(1, HID)] * B,
                             axis=1)
        bc = jnp.concatenate([bc_ref[...].reshape(1, HID)] * B, axis=1)
        u = jax.nn.sigmoid(uc[:, 0:FB] + bu)
        c = jnp.tanh(uc[:, FB:2 * FB] + bc)
        o = (1.0 - u) * c
        for b in range(B):
            out_ref[b, :, :] = o[:, HID * b:HID * b + HID].T


def kernel(inputs, adj, W_emb, b_emb, W_ru, b_ru, W_c, b_c):
    out_t = pl.pallas_call(
        _dcgru_body,
        grid=(2, NT),
        in_specs=[
            pl.BlockSpec((WINDOW, B, N), lambda k, i: (0, 0, 0)),
            pl.BlockSpec((TILE, N), lambda k, i: (i * (1 - k), 0)),
            pl.BlockSpec((WINDOW, EMB), lambda k, i: (0, 0)),
            pl.BlockSpec((2 * HID, (K + 1) * DIN), lambda k, i: (0, 0)),
            pl.BlockSpec((HID, (K + 1) * DIN), lambda k, i: (0, 0)),
            pl.BlockSpec((EMB,), lambda k, i: (0,)),
            pl.BlockSpec((2 * HID,), lambda k, i: (0,)),
            pl.BlockSpec((HID,), lambda k, i: (0,)),
        ],
        out_specs=pl.BlockSpec((B, HID, TILE), lambda k, i: (0, 0, i)),
        out_shape=jax.ShapeDtypeStruct((B, HID, N), jnp.float32),
        scratch_shapes=[
            pltpu.VMEM((N, FB), jnp.float32),
            pltpu.VMEM((N, FB), jnp.float32),
            pltpu.VMEM((N, N), jnp.bfloat16),
            pltpu.VMEM((N, FB), jnp.bfloat16),
            pltpu.VMEM((N, FB), jnp.bfloat16),
            pltpu.VMEM((3 * FB, 2 * FB), jnp.float32),
        ],
        compiler_params=pltpu.CompilerParams(
            dimension_semantics=("arbitrary", "arbitrary")),
    )(inputs.transpose(2, 0, 1), adj, W_emb, W_ru.T, W_c.T,
      b_emb, b_ru, b_c)
    return out_t.transpose(0, 2, 1)


# speedup vs baseline: 2.1179x; 1.1492x over previous
import jax
import jax.numpy as jnp
from jax.experimental import pallas as pl
from jax.experimental.pallas import tpu as pltpu

B = 4
N = 2048
WINDOW = 12
EMB = 32
HID = 32
K = 2
DIN = EMB + HID
FB = B * EMB
TILE = 512
NT = N // TILE


def _dcgru_body(inp_ref, adj_ref, wemb_ref, wrut_ref, wct_ref,
                bemb_ref, bru_ref, bc_ref, out_ref,
                x_sc, x1_sc, adj_sc, xb_sc, x1b_sc, wuc_sc):
    k = pl.program_id(0)
    i = pl.program_id(1)

    @pl.when((k == 0) & (i == 0))
    def _setup():
        wuc_sc[...] = jnp.zeros_like(wuc_sc)
        for s in range(K + 1):
            for b in range(B):
                r0 = FB * s + EMB * b
                wuc_sc[r0:r0 + EMB, HID * b:HID * b + HID] = (
                    wrut_ref[HID:2 * HID, DIN * s:DIN * s + EMB].T)
                wuc_sc[r0:r0 + EMB, FB + HID * b:FB + HID * b + HID] = (
                    wct_ref[:, DIN * s:DIN * s + EMB].T)
        for b in range(B):
            xtb = jax.lax.dot_general(
                wemb_ref[...], inp_ref[:, b, :],
                (((0,), (0,)), ((), ())),
                preferred_element_type=jnp.float32)
            x_sc[:, EMB * b:EMB * b + EMB] = (
                xtb.T + bemb_ref[...].reshape(1, EMB))
        xb_sc[...] = x_sc[...].astype(jnp.bfloat16)

    @pl.when(k == 0)
    def _pass1():
        a = adj_ref[...]
        ab = a.astype(jnp.bfloat16)
        adj_sc[pl.ds(i * TILE, TILE), :] = ab
        rs = jnp.sum(a, axis=1, keepdims=True) + 1e-8
        y = jnp.dot(ab, xb_sc[...], preferred_element_type=jnp.float32)
        x1 = y / rs
        x1_sc[pl.ds(i * TILE, TILE), :] = x1
        x1b_sc[pl.ds(i * TILE, TILE), :] = x1.astype(jnp.bfloat16)

    @pl.when(k == 1)
    def _pass2():
        a = adj_sc[pl.ds(i * TILE, TILE), :]
        rs = (jnp.sum(a.astype(jnp.float32), axis=1, keepdims=True) + 1e-8)
        x2 = jnp.dot(a, x1b_sc[...], preferred_element_type=jnp.float32) / rs
        xt = x_sc[pl.ds(i * TILE, TILE), :]
        x1t = x1_sc[pl.ds(i * TILE, TILE), :]
        uc = (jnp.dot(xt, wuc_sc[0:FB, :], preferred_element_type=jnp.float32)
              + jnp.dot(x1t, wuc_sc[FB:2 * FB, :],
                        preferred_element_type=jnp.float32)
              + jnp.dot(x2, wuc_sc[2 * FB:3 * FB, :],
                        preferred_element_type=jnp.float32))
        bu = jnp.concatenate([bru_ref[HID:2 * HID].reshape(1, HID)] * B,
                             axis=1)
        bc = jnp.concatenate([bc_ref[...].reshape(1, HID)] * B, axis=1)
        u = jax.nn.sigmoid(uc[:, 0:FB] + bu)
        c = jnp.tanh(uc[:, FB:2 * FB] + bc)
        o = (1.0 - u) * c
        for b in range(B):
            out_ref[b, :, :] = o[:, HID * b:HID * b + HID].T


def kernel(inputs, adj, W_emb, b_emb, W_ru, b_ru, W_c, b_c):
    out_t = pl.pallas_call(
        _dcgru_body,
        grid=(2, NT),
        in_specs=[
            pl.BlockSpec((WINDOW, B, N), lambda k, i: (0, 0, 0)),
            pl.BlockSpec((TILE, N), lambda k, i: (i * (1 - k), 0)),
            pl.BlockSpec((WINDOW, EMB), lambda k, i: (0, 0)),
            pl.BlockSpec((2 * HID, (K + 1) * DIN), lambda k, i: (0, 0)),
            pl.BlockSpec((HID, (K + 1) * DIN), lambda k, i: (0, 0)),
            pl.BlockSpec((EMB,), lambda k, i: (0,)),
            pl.BlockSpec((2 * HID,), lambda k, i: (0,)),
            pl.BlockSpec((HID,), lambda k, i: (0,)),
        ],
        out_specs=pl.BlockSpec((B, HID, TILE), lambda k, i: (0, 0, i)),
        out_shape=jax.ShapeDtypeStruct((B, HID, N), jnp.float32),
        scratch_shapes=[
            pltpu.VMEM((N, FB), jnp.float32),
            pltpu.VMEM((N, FB), jnp.float32),
            pltpu.VMEM((N, N), jnp.bfloat16),
            pltpu.VMEM((N, FB), jnp.bfloat16),
            pltpu.VMEM((N, FB), jnp.bfloat16),
            pltpu.VMEM((3 * FB, 2 * FB), jnp.float32),
        ],
        compiler_params=pltpu.CompilerParams(
            dimension_semantics=("arbitrary", "arbitrary")),
    )(inputs.transpose(2, 0, 1), adj, W_emb, W_ru.T, W_c.T,
      b_emb, b_ru, b_c)
    return out_t.transpose(0, 2, 1)


# speedup vs baseline: 2.1652x; 1.0223x over previous
import jax
import jax.numpy as jnp
from jax.experimental import pallas as pl
from jax.experimental.pallas import tpu as pltpu

B = 4
N = 2048
WINDOW = 12
EMB = 32
HID = 32
K = 2
DIN = EMB + HID
FB = B * EMB
TILE = 1024
NT = N // TILE


def _dcgru_body(inp_ref, adj_ref, wemb_ref, wrut_ref, wct_ref,
                bemb_ref, bru_ref, bc_ref, out_ref,
                x_sc, x1_sc, adj_sc, xb_sc, x1b_sc, wuc_sc):
    k = pl.program_id(0)
    i = pl.program_id(1)

    @pl.when((k == 0) & (i == 0))
    def _setup():
        wuc_sc[...] = jnp.zeros_like(wuc_sc)
        for s in range(K + 1):
            for b in range(B):
                r0 = FB * s + EMB * b
                wuc_sc[r0:r0 + EMB, HID * b:HID * b + HID] = (
                    wrut_ref[HID:2 * HID, DIN * s:DIN * s + EMB].T)
                wuc_sc[r0:r0 + EMB, FB + HID * b:FB + HID * b + HID] = (
                    wct_ref[:, DIN * s:DIN * s + EMB].T)
        for b in range(B):
            xtb = jax.lax.dot_general(
                wemb_ref[...], inp_ref[:, b, :],
                (((0,), (0,)), ((), ())),
                preferred_element_type=jnp.float32)
            x_sc[:, EMB * b:EMB * b + EMB] = (
                xtb.T + bemb_ref[...].reshape(1, EMB))
        xb_sc[...] = x_sc[...].astype(jnp.bfloat16)

    @pl.when(k == 0)
    def _pass1():
        a = adj_ref[...]
        ab = a.astype(jnp.bfloat16)
        adj_sc[pl.ds(i * TILE, TILE), :] = ab
        rs = jnp.sum(a, axis=1, keepdims=True) + 1e-8
        y = jnp.dot(ab, xb_sc[...], preferred_element_type=jnp.float32)
        x1 = y / rs
        x1_sc[pl.ds(i * TILE, TILE), :] = x1
        x1b_sc[pl.ds(i * TILE, TILE), :] = x1.astype(jnp.bfloat16)

    @pl.when(k == 1)
    def _pass2():
        a = adj_sc[pl.ds(i * TILE, TILE), :]
        rs = (jnp.sum(a.astype(jnp.float32), axis=1, keepdims=True) + 1e-8)
        x2 = jnp.dot(a, x1b_sc[...], preferred_element_type=jnp.float32) / rs
        xt = x_sc[pl.ds(i * TILE, TILE), :]
        x1t = x1_sc[pl.ds(i * TILE, TILE), :]
        uc = (jnp.dot(xt, wuc_sc[0:FB, :], preferred_element_type=jnp.float32)
              + jnp.dot(x1t, wuc_sc[FB:2 * FB, :],
                        preferred_element_type=jnp.float32)
              + jnp.dot(x2, wuc_sc[2 * FB:3 * FB, :],
                        preferred_element_type=jnp.float32))
        bu = jnp.concatenate([bru_ref[HID:2 * HID].reshape(1, HID)] * B,
                             axis=1)
        bc = jnp.concatenate([bc_ref[...].reshape(1, HID)] * B, axis=1)
        u = jax.nn.sigmoid(uc[:, 0:FB] + bu)
        c = jnp.tanh(uc[:, FB:2 * FB] + bc)
        o = (1.0 - u) * c
        for b in range(B):
            out_ref[b, :, :] = o[:, HID * b:HID * b + HID].T


def kernel(inputs, adj, W_emb, b_emb, W_ru, b_ru, W_c, b_c):
    out_t = pl.pallas_call(
        _dcgru_body,
        grid=(2, NT),
        in_specs=[
            pl.BlockSpec((WINDOW, B, N), lambda k, i: (0, 0, 0)),
            pl.BlockSpec((TILE, N), lambda k, i: (i * (1 - k), 0)),
            pl.BlockSpec((WINDOW, EMB), lambda k, i: (0, 0)),
            pl.BlockSpec((2 * HID, (K + 1) * DIN), lambda k, i: (0, 0)),
            pl.BlockSpec((HID, (K + 1) * DIN), lambda k, i: (0, 0)),
            pl.BlockSpec((EMB,), lambda k, i: (0,)),
            pl.BlockSpec((2 * HID,), lambda k, i: (0,)),
            pl.BlockSpec((HID,), lambda k, i: (0,)),
        ],
        out_specs=pl.BlockSpec((B, HID, TILE), lambda k, i: (0, 0, i)),
        out_shape=jax.ShapeDtypeStruct((B, HID, N), jnp.float32),
        scratch_shapes=[
            pltpu.VMEM((N, FB), jnp.float32),
            pltpu.VMEM((N, FB), jnp.float32),
            pltpu.VMEM((N, N), jnp.bfloat16),
            pltpu.VMEM((N, FB), jnp.bfloat16),
            pltpu.VMEM((N, FB), jnp.bfloat16),
            pltpu.VMEM((3 * FB, 2 * FB), jnp.float32),
        ],
        compiler_params=pltpu.CompilerParams(
            dimension_semantics=("arbitrary", "arbitrary")),
    )(inputs.transpose(2, 0, 1), adj, W_emb, W_ru.T, W_c.T,
      b_emb, b_ru, b_c)
    return out_t.transpose(0, 2, 1)


# speedup vs baseline: 2.2254x; 1.0278x over previous
import jax
import jax.numpy as jnp
from jax.experimental import pallas as pl
from jax.experimental.pallas import tpu as pltpu

B = 4
N = 2048
WINDOW = 12
EMB = 32
HID = 32
K = 2
DIN = EMB + HID
FB = B * EMB
CHUNK = 64
NC = N // CHUNK
TILE2 = 512
NT2 = N // TILE2


def _dcgru_body(inp_ref, adj_ref, wemb_ref, wrut_ref, wct_ref,
                bemb_ref, bru_ref, bc_ref, out_ref,
                x_sc, x1_sc, adj_sc, xb_sc, x1b_sc, wuc_sc, sems):
    for c in range(NC):
        pltpu.make_async_copy(
            adj_ref.at[pl.ds(c * CHUNK, CHUNK), :],
            adj_sc.at[pl.ds(c * CHUNK, CHUNK), :],
            sems.at[c]).start()

    wuc_sc[...] = jnp.zeros_like(wuc_sc)
    for s in range(K + 1):
        for b in range(B):
            r0 = FB * s + EMB * b
            wuc_sc[r0:r0 + EMB, HID * b:HID * b + HID] = (
                wrut_ref[HID:2 * HID, DIN * s:DIN * s + EMB].T)
            wuc_sc[r0:r0 + EMB, FB + HID * b:FB + HID * b + HID] = (
                wct_ref[:, DIN * s:DIN * s + EMB].T)
    for b in range(B):
        xtb = jax.lax.dot_general(
            wemb_ref[...], inp_ref[:, b, :],
            (((0,), (0,)), ((), ())),
            preferred_element_type=jnp.float32)
        x_sc[:, EMB * b:EMB * b + EMB] = (
            xtb.T + bemb_ref[...].reshape(1, EMB))
    xb_sc[...] = x_sc[...].astype(jnp.bfloat16)

    for c in range(NC):
        pltpu.make_async_copy(
            adj_ref.at[pl.ds(c * CHUNK, CHUNK), :],
            adj_sc.at[pl.ds(c * CHUNK, CHUNK), :],
            sems.at[c]).wait()
        a = adj_sc[pl.ds(c * CHUNK, CHUNK), :]
        rs = jnp.sum(a, axis=1, keepdims=True) + 1e-8
        x1 = jnp.dot(a.astype(jnp.bfloat16), xb_sc[...],
                     preferred_element_type=jnp.float32) / rs
        x1_sc[pl.ds(c * CHUNK, CHUNK), :] = x1
        x1b_sc[pl.ds(c * CHUNK, CHUNK), :] = x1.astype(jnp.bfloat16)

    for t in range(NT2):
        a = adj_sc[pl.ds(t * TILE2, TILE2), :]
        rs = jnp.sum(a, axis=1, keepdims=True) + 1e-8
        x2 = jnp.dot(a.astype(jnp.bfloat16), x1b_sc[...],
                     preferred_element_type=jnp.float32) / rs
        xt = x_sc[pl.ds(t * TILE2, TILE2), :]
        x1t = x1_sc[pl.ds(t * TILE2, TILE2), :]
        uc = (jnp.dot(xt, wuc_sc[0:FB, :], preferred_element_type=jnp.float32)
              + jnp.dot(x1t, wuc_sc[FB:2 * FB, :],
                        preferred_element_type=jnp.float32)
              + jnp.dot(x2, wuc_sc[2 * FB:3 * FB, :],
                        preferred_element_type=jnp.float32))
        bu = jnp.concatenate([bru_ref[HID:2 * HID].reshape(1, HID)] * B,
                             axis=1)
        bc = jnp.concatenate([bc_ref[...].reshape(1, HID)] * B, axis=1)
        u = jax.nn.sigmoid(uc[:, 0:FB] + bu)
        c2 = jnp.tanh(uc[:, FB:2 * FB] + bc)
        o = (1.0 - u) * c2
        for b in range(B):
            out_ref[b, :, pl.ds(t * TILE2, TILE2)] = (
                o[:, HID * b:HID * b + HID].T)


def kernel(inputs, adj, W_emb, b_emb, W_ru, b_ru, W_c, b_c):
    out_t = pl.pallas_call(
        _dcgru_body,
        in_specs=[
            pl.BlockSpec((WINDOW, B, N), lambda: (0, 0, 0)),
            pl.BlockSpec(memory_space=pl.ANY),
            pl.BlockSpec((WINDOW, EMB), lambda: (0, 0)),
            pl.BlockSpec((2 * HID, (K + 1) * DIN), lambda: (0, 0)),
            pl.BlockSpec((HID, (K + 1) * DIN), lambda: (0, 0)),
            pl.BlockSpec((EMB,), lambda: (0,)),
            pl.BlockSpec((2 * HID,), lambda: (0,)),
            pl.BlockSpec((HID,), lambda: (0,)),
        ],
        out_specs=pl.BlockSpec((B, HID, N), lambda: (0, 0, 0)),
        out_shape=jax.ShapeDtypeStruct((B, HID, N), jnp.float32),
        scratch_shapes=[
            pltpu.VMEM((N, FB), jnp.float32),
            pltpu.VMEM((N, FB), jnp.float32),
            pltpu.VMEM((N, N), jnp.float32),
            pltpu.VMEM((N, FB), jnp.bfloat16),
            pltpu.VMEM((N, FB), jnp.bfloat16),
            pltpu.VMEM((3 * FB, 2 * FB), jnp.float32),
            pltpu.SemaphoreType.DMA((NC,)),
        ],
        compiler_params=pltpu.CompilerParams(),
    )(inputs.transpose(2, 0, 1), adj, W_emb, W_ru.T, W_c.T,
      b_emb, b_ru, b_c)
    return out_t.transpose(0, 2, 1)
